# baseline (speedup 1.0000x reference)
import jax
import jax.numpy as jnp
from jax import lax
from jax.experimental import pallas as pl
from jax.experimental.pallas import tpu as pltpu
from jax.experimental.pallas import tpu_sc as plsc

NUM_CORES = 2
NUM_SUBCORES = 16
LANES = 16
NUM_WORKERS = NUM_CORES * NUM_SUBCORES

BATCH = 16384
EMBED_DIM = 32
TILE_W = 128
B_PER_W = BATCH // NUM_WORKERS
CHUNK = 16
NUM_CHUNKS = B_PER_W // CHUNK
K_I = 14


def _sc_kernel(user_ids_hbm, item_ids_hbm, ut_hbm, it_hbm,
               out_hbm, idxu_v, idxi_v, wbuf, ibuf, uval, out_v,
               sem_u, sem_i):
    wid = lax.axis_index("s") * NUM_CORES + lax.axis_index("c")
    base = wid * B_PER_W

    pltpu.sync_copy(user_ids_hbm.at[pl.ds(base, B_PER_W)], idxu_v)
    pltpu.sync_copy(item_ids_hbm.at[pl.ds(base, B_PER_W)], idxi_v)

    lanes16 = lax.iota(jnp.int32, LANES)
    row_of_lane = lanes16 * EMBED_DIM
    row_item = jnp.minimum(lanes16, K_I - 1) * EMBED_DIM
    row_tail = jnp.clip(lanes16 - K_I, 0, CHUNK - K_I - 1) * EMBED_DIM

    def window(tbl_hbm, idx):
        col = pl.multiple_of((idx >> 7) << 7, TILE_W)
        return tbl_hbm.at[:, pl.ds(col, TILE_W)]

    def chunk_body(cb, carry):
        c0 = cb * CHUNK
        uvec = idxu_v[pl.ds(c0, CHUNK)]
        ivec = idxi_v[pl.ds(c0, CHUNK)]

        ucps = [pltpu.async_copy(
                    window(ut_hbm, uvec[q]),
                    wbuf.at[pl.ds(q * EMBED_DIM, EMBED_DIM), :], sem_u)
                for q in range(CHUNK)]
        icps = [pltpu.async_copy(
                    window(it_hbm, ivec[q]),
                    ibuf.at[pl.ds(q * EMBED_DIM, EMBED_DIM), :], sem_i)
                for q in range(K_I)]

        for cp in ucps:
            cp.wait()
        lane_u = uvec & (TILE_W - 1)
        for d in range(EMBED_DIM):
            uval[d, :] = plsc.load_gather(wbuf, [row_of_lane + d, lane_u])

        tail = [pltpu.async_copy(
                    window(it_hbm, ivec[q]),
                    wbuf.at[pl.ds((q - K_I) * EMBED_DIM, EMBED_DIM), :],
                    sem_i)
                for q in range(K_I, CHUNK)]
        for cp in icps:
            cp.wait()
        for cp in tail:
            cp.wait()

        lane_i = ivec & (TILE_W - 1)
        in_tail = lanes16 >= K_I
        acc = jnp.zeros((LANES,), jnp.float32)
        for d in range(EMBED_DIM):
            gm = plsc.load_gather(ibuf, [row_item + d, lane_i])
            gl = plsc.load_gather(wbuf, [row_tail + d, lane_i])
            acc = acc + uval[d, :] * jnp.where(in_tail, gl, gm)
        out_v[pl.ds(c0, CHUNK)] = acc
        return carry

    lax.fori_loop(0, NUM_CHUNKS, chunk_body, 0)

    pltpu.sync_copy(out_v, out_hbm.at[pl.ds(base, B_PER_W)])


@jax.jit
def kernel(user_ids, item_ids, user_table, item_table):
    mesh = plsc.VectorSubcoreMesh(
        core_axis_name="c", subcore_axis_name="s",
        num_cores=NUM_CORES, num_subcores=NUM_SUBCORES)
    run = pl.kernel(
        _sc_kernel,
        out_type=jax.ShapeDtypeStruct((BATCH,), jnp.float32),
        mesh=mesh,
        scratch_types=[
            pltpu.VMEM((B_PER_W,), jnp.int32),
            pltpu.VMEM((B_PER_W,), jnp.int32),
            pltpu.VMEM((CHUNK * EMBED_DIM, TILE_W), jnp.float32),
            pltpu.VMEM((K_I * EMBED_DIM, TILE_W), jnp.float32),
            pltpu.VMEM((EMBED_DIM, LANES), jnp.float32),
            pltpu.VMEM((B_PER_W,), jnp.float32),
            pltpu.SemaphoreType.DMA,
            pltpu.SemaphoreType.DMA,
        ],
        compiler_params=pltpu.CompilerParams(needs_layout_passes=False),
    )
    return run(user_ids.astype(jnp.int32), item_ids.astype(jnp.int32),
               user_table.T, item_table.T)
